# baseline (speedup 1.0000x reference)
import functools

import jax
import jax.numpy as jnp
from jax import lax
from jax.experimental import pallas as pl
from jax.experimental.pallas import tpu as pltpu
from jax.experimental.pallas import tpu_sc as plsc

N = 20
D = 32
P = N * (N - 1) // 2
NP = N + P
ROW_W = NP * D
L = 16
NC, NS = 2, 16
NW = NC * NS

DETILE_CH = 8192
DETILE_Q = 128 // D
DETILE_CH4 = DETILE_CH // DETILE_Q

SC_CHUNK = 1024
SC_CHUNKS_PER = 26
WSC = NW * SC_CHUNKS_PER * SC_CHUNK
BLK0 = WSC // DETILE_CH


@functools.lru_cache(maxsize=None)
def _build_tc_detile(v: int):
    g = -(-v // DETILE_CH) - BLK0

    def body(x_ref, o_ref):
        x = x_ref[...]
        y = jnp.concatenate(
            [x[:, q * DETILE_CH4:(q + 1) * DETILE_CH4]
             for q in range(DETILE_Q)], axis=0)
        o_ref[...] = y.T

    return pl.pallas_call(
        body,
        grid=(g,),
        in_specs=[pl.BlockSpec((D, DETILE_CH), lambda i: (0, i + BLK0))],
        out_specs=pl.BlockSpec((DETILE_CH4, 128), lambda i: (i, 0)),
        out_shape=jax.ShapeDtypeStruct((g * DETILE_CH4, 128), jnp.float32),
    )


@functools.lru_cache(maxsize=None)
def _build_sc_detile(v: int):
    mesh = plsc.VectorSubcoreMesh(core_axis_name="c", subcore_axis_name="s")

    @functools.partial(
        pl.kernel,
        out_type=jax.ShapeDtypeStruct((D * WSC,), jnp.float32),
        mesh=mesh,
        scratch_types=[
            pltpu.VMEM((D, SC_CHUNK), jnp.float32),
            pltpu.VMEM((D, SC_CHUNK), jnp.float32),
            pltpu.SemaphoreType.DMA,
            pltpu.SemaphoreType.DMA,
            pltpu.SemaphoreType.DMA,
        ],
    )
    def sc_detile(table_hbm, scr_hbm, buf0, buf1, isem, osem0, osem1):
        wid = lax.axis_index("s") * NC + lax.axis_index("c")
        cbase = wid * (SC_CHUNKS_PER * SC_CHUNK)

        bufs = (buf0, buf1)
        osems = (osem0, osem1)

        def chunk_pair(t, carry):
            for b in range(2):
                buf = bufs[b]
                osem = osems[b]
                c = cbase + (t * 2 + b) * SC_CHUNK
                c = pl.multiple_of(c, 128)

                @pl.when(t > 0)
                def _wait_prev():
                    for _ in range(D):
                        pltpu.make_async_copy(
                            buf.at[0], scr_hbm.at[pl.ds(0, SC_CHUNK)],
                            osem).wait()

                pltpu.sync_copy(table_hbm.at[:, pl.ds(c, SC_CHUNK)], buf)
                for d in range(D):
                    off = pl.multiple_of(d * WSC + c, 8)
                    pltpu.async_copy(
                        buf.at[d], scr_hbm.at[pl.ds(off, SC_CHUNK)], osem)
            return carry

        lax.fori_loop(0, SC_CHUNKS_PER // 2, chunk_pair, 0)

        for b in range(2):
            for _ in range(D):
                pltpu.make_async_copy(
                    bufs[b].at[0], scr_hbm.at[pl.ds(0, SC_CHUNK)],
                    osems[b]).wait()

    return sc_detile


@functools.lru_cache(maxsize=None)
def _build_main(batch: int):
    assert batch % NW == 0
    bpw = batch // NW
    n_idx = bpw * N

    mesh = plsc.VectorSubcoreMesh(core_axis_name="c", subcore_axis_name="s")

    @functools.partial(
        pl.kernel,
        out_type=jax.ShapeDtypeStruct((batch, ROW_W), jnp.float32),
        mesh=mesh,
        scratch_types=[
            pltpu.VMEM((bpw, 2 * L), jnp.int32),
            pltpu.VMEM((n_idx, D), jnp.float32),
            pltpu.VMEM((2 * L,), jnp.float32),
            pltpu.VMEM((N, D), jnp.float32),
            pltpu.VMEM((ROW_W,), jnp.float32),
            pltpu.VMEM((ROW_W,), jnp.float32),
            pltpu.SemaphoreType.DMA,
            pltpu.SemaphoreType.DMA,
            pltpu.SemaphoreType.DMA,
        ],
    )
    def sc_kernel(mem_hbm, scr_sc_hbm, scr_tc_hbm, w_hbm, out_hbm,
                  idx_v, rows_v, w_v, srow_v, obuf0, obuf1,
                  gsem, osem0, osem1):
        wid = lax.axis_index("s") * NC + lax.axis_index("c")
        base = wid * bpw

        pltpu.sync_copy(w_hbm, w_v.at[pl.ds(0, N)])

        def stage_idx(r, carry):
            pltpu.async_copy(mem_hbm.at[base + r],
                             idx_v.at[r, pl.ds(0, N)], gsem)
            return carry

        lax.fori_loop(0, bpw, stage_idx, 0)

        def stage_drain(r, carry):
            pltpu.make_async_copy(mem_hbm.at[0],
                                  idx_v.at[0, pl.ds(0, N)], gsem).wait()
            return carry

        lax.fori_loop(0, bpw, stage_drain, 0)

        iota = lax.iota(jnp.int32, L)

        def gather_row(r, carry):
            v0 = idx_v[r, pl.ds(0, L)]
            v1 = idx_v[r, pl.ds(L, L)]
            for k in range(N):
                row = v0[k] if k < L else v1[k - L]
                slot = r * N + k

                @pl.when(row < WSC)
                def _from_sc():
                    idx0 = iota * WSC + row
                    pltpu.async_copy(scr_sc_hbm.at[idx0],
                                     rows_v.at[slot, pl.ds(0, L)], gsem)
                    pltpu.async_copy(scr_sc_hbm.at[idx0 + (L * WSC)],
                                     rows_v.at[slot, pl.ds(L, L)], gsem)

                @pl.when(row >= WSC)
                def _from_tc():
                    lcl = row - WSC
                    off = (((lcl >> 13) << 18) + ((lcl & 2047) << 7)
                           + (((lcl >> 11) & 3) << 5))
                    off = pl.multiple_of(off, 32)
                    pltpu.async_copy(scr_tc_hbm.at[pl.ds(off, D)],
                                     rows_v.at[slot], gsem)
            return carry

        lax.fori_loop(0, bpw, gather_row, 0)

        def gather_drain(r, carry):
            for k in range(N):
                pltpu.make_async_copy(scr_tc_hbm.at[pl.ds(0, D)],
                                      rows_v.at[0], gsem).wait()
            return carry

        lax.fori_loop(0, bpw, gather_drain, 0)

        wv0 = w_v[pl.ds(0, L)]
        wv1 = w_v[pl.ds(L, L)]
        w_sc = [wv0[i] if i < L else wv1[i - L] for i in range(N)]

        obufs = (obuf0, obuf1)
        osems = (osem0, osem1)

        def row_pair_body(t, carry):
            for b in range(2):
                r = t * 2 + b
                obuf = obufs[b]
                osem = osems[b]
                lbase = r * N

                @pl.when(t > 0)
                def _wait_prev():
                    pltpu.make_async_copy(
                        obuf, out_hbm.at[0], osem).wait()

                for i in range(N):
                    for h in range(2):
                        sl = pl.ds(h * L, L)
                        v = rows_v[lbase + i, sl]
                        obuf[pl.ds(i * D + h * L, L)] = v
                        srow_v[i, sl] = v * w_sc[i]

                p = 0
                for i in range(N):
                    si0 = srow_v[i, pl.ds(0, L)]
                    si1 = srow_v[i, pl.ds(L, L)]
                    for j in range(i + 1, N):
                        o = (N + p) * D
                        obuf[pl.ds(o, L)] = si0 * srow_v[j, pl.ds(0, L)]
                        obuf[pl.ds(o + L, L)] = si1 * srow_v[j, pl.ds(L, L)]
                        p += 1

                pltpu.async_copy(obuf, out_hbm.at[base + r], osem)
            return carry

        lax.fori_loop(0, bpw // 2, row_pair_body, 0)

        pltpu.make_async_copy(obuf0, out_hbm.at[0], osem0).wait()
        pltpu.make_async_copy(obuf1, out_hbm.at[0], osem1).wait()

    return sc_kernel


def kernel(user, memory, item_table, weights):
    del user
    batch = memory.shape[0]
    v = item_table.shape[0]
    table_t = item_table.T
    scr_sc = _build_sc_detile(v)(table_t)
    scr_tc = _build_tc_detile(v)(table_t).reshape(-1)
    sc_kernel = _build_main(batch)
    return sc_kernel(memory.astype(jnp.int32), scr_sc, scr_tc, weights)


# speedup vs baseline: 1.0964x; 1.0964x over previous
import functools

import jax
import jax.numpy as jnp
from jax import lax
from jax.experimental import pallas as pl
from jax.experimental.pallas import tpu as pltpu
from jax.experimental.pallas import tpu_sc as plsc

N = 20
D = 32
P = N * (N - 1) // 2
NP = N + P
ROW_W = NP * D
L = 16
NC, NS = 2, 16
NW = NC * NS

DETILE_CH = 8192
DETILE_Q = 128 // D
DETILE_CH4 = DETILE_CH // DETILE_Q

SC_CHUNK = 1024
SC_CHUNKS_PER = 18
WSC = NW * SC_CHUNKS_PER * SC_CHUNK
BLK0 = WSC // DETILE_CH


@functools.lru_cache(maxsize=None)
def _build_tc_detile(v: int):
    g = -(-v // DETILE_CH) - BLK0

    def body(x_ref, o_ref):
        x = x_ref[...]
        y = jnp.concatenate(
            [x[:, q * DETILE_CH4:(q + 1) * DETILE_CH4]
             for q in range(DETILE_Q)], axis=0)
        o_ref[...] = y.T

    return pl.pallas_call(
        body,
        grid=(g,),
        in_specs=[pl.BlockSpec((D, DETILE_CH), lambda i: (0, i + BLK0))],
        out_specs=pl.BlockSpec((DETILE_CH4, 128), lambda i: (i, 0)),
        out_shape=jax.ShapeDtypeStruct((g * DETILE_CH4, 128), jnp.float32),
    )


@functools.lru_cache(maxsize=None)
def _build_sc_detile(v: int):
    mesh = plsc.VectorSubcoreMesh(core_axis_name="c", subcore_axis_name="s")

    @functools.partial(
        pl.kernel,
        out_type=jax.ShapeDtypeStruct((D * WSC,), jnp.float32),
        mesh=mesh,
        scratch_types=[
            pltpu.VMEM((D, SC_CHUNK), jnp.float32),
            pltpu.VMEM((D, SC_CHUNK), jnp.float32),
            pltpu.SemaphoreType.DMA,
            pltpu.SemaphoreType.DMA,
            pltpu.SemaphoreType.DMA,
        ],
    )
    def sc_detile(table_hbm, scr_hbm, buf0, buf1, isem, osem0, osem1):
        wid = lax.axis_index("s") * NC + lax.axis_index("c")
        cbase = wid * (SC_CHUNKS_PER * SC_CHUNK)

        bufs = (buf0, buf1)
        osems = (osem0, osem1)

        def chunk_pair(t, carry):
            for b in range(2):
                buf = bufs[b]
                osem = osems[b]
                c = cbase + (t * 2 + b) * SC_CHUNK
                c = pl.multiple_of(c, 128)

                @pl.when(t > 0)
                def _wait_prev():
                    for _ in range(D):
                        pltpu.make_async_copy(
                            buf.at[0], scr_hbm.at[pl.ds(0, SC_CHUNK)],
                            osem).wait()

                pltpu.sync_copy(table_hbm.at[:, pl.ds(c, SC_CHUNK)], buf)
                for d in range(D):
                    off = pl.multiple_of(d * WSC + c, 8)
                    pltpu.async_copy(
                        buf.at[d], scr_hbm.at[pl.ds(off, SC_CHUNK)], osem)
            return carry

        lax.fori_loop(0, SC_CHUNKS_PER // 2, chunk_pair, 0)

        for b in range(2):
            for _ in range(D):
                pltpu.make_async_copy(
                    bufs[b].at[0], scr_hbm.at[pl.ds(0, SC_CHUNK)],
                    osems[b]).wait()

    return sc_detile


@functools.lru_cache(maxsize=None)
def _build_main(batch: int):
    assert batch % NW == 0
    bpw = batch // NW
    n_idx = bpw * N

    mesh = plsc.VectorSubcoreMesh(core_axis_name="c", subcore_axis_name="s")

    @functools.partial(
        pl.kernel,
        out_type=jax.ShapeDtypeStruct((batch, ROW_W), jnp.float32),
        mesh=mesh,
        scratch_types=[
            pltpu.VMEM((bpw, 2 * L), jnp.int32),
            pltpu.VMEM((n_idx, D), jnp.float32),
            pltpu.VMEM((2 * L,), jnp.float32),
            pltpu.VMEM((N, D), jnp.float32),
            pltpu.VMEM((ROW_W,), jnp.float32),
            pltpu.VMEM((ROW_W,), jnp.float32),
            pltpu.SemaphoreType.DMA,
            pltpu.SemaphoreType.DMA,
            pltpu.SemaphoreType.DMA,
        ],
    )
    def sc_kernel(mem_hbm, scr_sc_hbm, scr_tc_hbm, w_hbm, out_hbm,
                  idx_v, rows_v, w_v, srow_v, obuf0, obuf1,
                  gsem, osem0, osem1):
        wid = lax.axis_index("s") * NC + lax.axis_index("c")
        base = wid * bpw

        pltpu.sync_copy(w_hbm, w_v.at[pl.ds(0, N)])

        def stage_idx(r, carry):
            pltpu.async_copy(mem_hbm.at[base + r],
                             idx_v.at[r, pl.ds(0, N)], gsem)
            return carry

        lax.fori_loop(0, bpw, stage_idx, 0)

        def stage_drain(r, carry):
            pltpu.make_async_copy(mem_hbm.at[0],
                                  idx_v.at[0, pl.ds(0, N)], gsem).wait()
            return carry

        lax.fori_loop(0, bpw, stage_drain, 0)

        iota = lax.iota(jnp.int32, L)

        def gather_row(r, carry):
            v0 = idx_v[r, pl.ds(0, L)]
            v1 = idx_v[r, pl.ds(L, L)]
            for k in range(N):
                row = v0[k] if k < L else v1[k - L]
                slot = r * N + k

                @pl.when(row < WSC)
                def _from_sc():
                    idx0 = iota * WSC + row
                    pltpu.async_copy(scr_sc_hbm.at[idx0],
                                     rows_v.at[slot, pl.ds(0, L)], gsem)
                    pltpu.async_copy(scr_sc_hbm.at[idx0 + (L * WSC)],
                                     rows_v.at[slot, pl.ds(L, L)], gsem)

                @pl.when(row >= WSC)
                def _from_tc():
                    lcl = row - WSC
                    off = (((lcl >> 13) << 18) + ((lcl & 2047) << 7)
                           + (((lcl >> 11) & 3) << 5))
                    off = pl.multiple_of(off, 32)
                    pltpu.async_copy(scr_tc_hbm.at[pl.ds(off, D)],
                                     rows_v.at[slot], gsem)
            return carry

        def gather_drain(r, carry):
            for k in range(N):
                pltpu.make_async_copy(scr_tc_hbm.at[pl.ds(0, D)],
                                      rows_v.at[0], gsem).wait()
            return carry

        half = bpw // 2
        lax.fori_loop(0, half, gather_row, 0)
        lax.fori_loop(0, half, gather_drain, 0)
        lax.fori_loop(half, bpw, gather_row, 0)

        wv0 = w_v[pl.ds(0, L)]
        wv1 = w_v[pl.ds(L, L)]
        w_sc = [wv0[i] if i < L else wv1[i - L] for i in range(N)]

        obufs = (obuf0, obuf1)
        osems = (osem0, osem1)

        def row_pair_body(t, carry):
            for b in range(2):
                r = t * 2 + b
                obuf = obufs[b]
                osem = osems[b]
                lbase = r * N

                @pl.when(t > 0)
                def _wait_prev():
                    pltpu.make_async_copy(
                        obuf, out_hbm.at[0], osem).wait()

                for i in range(N):
                    for h in range(2):
                        sl = pl.ds(h * L, L)
                        v = rows_v[lbase + i, sl]
                        obuf[pl.ds(i * D + h * L, L)] = v
                        srow_v[i, sl] = v * w_sc[i]

                p = 0
                for i in range(N):
                    si0 = srow_v[i, pl.ds(0, L)]
                    si1 = srow_v[i, pl.ds(L, L)]
                    for j in range(i + 1, N):
                        o = (N + p) * D
                        obuf[pl.ds(o, L)] = si0 * srow_v[j, pl.ds(0, L)]
                        obuf[pl.ds(o + L, L)] = si1 * srow_v[j, pl.ds(L, L)]
                        p += 1

                pltpu.async_copy(obuf, out_hbm.at[base + r], osem)
            return carry

        lax.fori_loop(0, bpw // 4, row_pair_body, 0)
        lax.fori_loop(0, half, gather_drain, 0)
        lax.fori_loop(bpw // 4, bpw // 2, row_pair_body, 0)

        pltpu.make_async_copy(obuf0, out_hbm.at[0], osem0).wait()
        pltpu.make_async_copy(obuf1, out_hbm.at[0], osem1).wait()

    return sc_kernel


def kernel(user, memory, item_table, weights):
    del user
    batch = memory.shape[0]
    v = item_table.shape[0]
    table_t = item_table.T
    scr_sc = _build_sc_detile(v)(table_t)
    scr_tc = _build_tc_detile(v)(table_t).reshape(-1)
    sc_kernel = _build_main(batch)
    return sc_kernel(memory.astype(jnp.int32), scr_sc, scr_tc, weights)


# speedup vs baseline: 1.1347x; 1.0349x over previous
import functools

import jax
import jax.numpy as jnp
from jax import lax
from jax.experimental import pallas as pl
from jax.experimental.pallas import tpu as pltpu
from jax.experimental.pallas import tpu_sc as plsc

N = 20
D = 32
P = N * (N - 1) // 2
NP = N + P
ROW_W = NP * D
L = 16
NC, NS = 2, 16
NW = NC * NS

DETILE_CH = 8192
DETILE_Q = 128 // D
DETILE_CH4 = DETILE_CH // DETILE_Q

SC_CHUNK = 1024
SC_CHUNKS_PER = 18
WSC = NW * SC_CHUNKS_PER * SC_CHUNK
BLK0 = WSC // DETILE_CH


@functools.lru_cache(maxsize=None)
def _build_tc_detile(v: int):
    g = -(-v // DETILE_CH) - BLK0

    def body(x_ref, o_ref):
        x = x_ref[...]
        y = jnp.concatenate(
            [x[:, q * DETILE_CH4:(q + 1) * DETILE_CH4]
             for q in range(DETILE_Q)], axis=0)
        o_ref[...] = y.T

    return pl.pallas_call(
        body,
        grid=(g,),
        in_specs=[pl.BlockSpec((D, DETILE_CH), lambda i: (0, i + BLK0))],
        out_specs=pl.BlockSpec((DETILE_CH4, 128), lambda i: (i, 0)),
        out_shape=jax.ShapeDtypeStruct((g * DETILE_CH4, 128), jnp.float32),
    )


@functools.lru_cache(maxsize=None)
def _build_sc_detile(v: int):
    mesh = plsc.VectorSubcoreMesh(core_axis_name="c", subcore_axis_name="s")

    @functools.partial(
        pl.kernel,
        out_type=jax.ShapeDtypeStruct((D * WSC,), jnp.float32),
        mesh=mesh,
        scratch_types=[
            pltpu.VMEM((D, SC_CHUNK), jnp.float32),
            pltpu.VMEM((D, SC_CHUNK), jnp.float32),
            pltpu.SemaphoreType.DMA,
            pltpu.SemaphoreType.DMA,
            pltpu.SemaphoreType.DMA,
        ],
    )
    def sc_detile(table_hbm, scr_hbm, buf0, buf1, isem, osem0, osem1):
        wid = lax.axis_index("s") * NC + lax.axis_index("c")
        cbase = wid * (SC_CHUNKS_PER * SC_CHUNK)

        bufs = (buf0, buf1)
        osems = (osem0, osem1)

        def chunk_pair(t, carry):
            for b in range(2):
                buf = bufs[b]
                osem = osems[b]
                c = cbase + (t * 2 + b) * SC_CHUNK
                c = pl.multiple_of(c, 128)

                @pl.when(t > 0)
                def _wait_prev():
                    for _ in range(D):
                        pltpu.make_async_copy(
                            buf.at[0], scr_hbm.at[pl.ds(0, SC_CHUNK)],
                            osem).wait()

                pltpu.sync_copy(table_hbm.at[:, pl.ds(c, SC_CHUNK)], buf)
                for d in range(D):
                    off = pl.multiple_of(d * WSC + c, 8)
                    pltpu.async_copy(
                        buf.at[d], scr_hbm.at[pl.ds(off, SC_CHUNK)], osem)
            return carry

        lax.fori_loop(0, SC_CHUNKS_PER // 2, chunk_pair, 0)

        for b in range(2):
            for _ in range(D):
                pltpu.make_async_copy(
                    bufs[b].at[0], scr_hbm.at[pl.ds(0, SC_CHUNK)],
                    osems[b]).wait()

    return sc_detile


@functools.lru_cache(maxsize=None)
def _build_main(batch: int):
    assert batch % NW == 0
    bpw = batch // NW
    n_idx = bpw * N

    mesh = plsc.VectorSubcoreMesh(core_axis_name="c", subcore_axis_name="s")

    @functools.partial(
        pl.kernel,
        out_type=jax.ShapeDtypeStruct((batch, ROW_W), jnp.float32),
        mesh=mesh,
        scratch_types=[
            pltpu.VMEM((bpw, 2 * L), jnp.int32),
            pltpu.VMEM((n_idx, D), jnp.float32),
            pltpu.VMEM((2 * L,), jnp.float32),
            pltpu.VMEM((N, D), jnp.float32),
            pltpu.VMEM((ROW_W,), jnp.float32),
            pltpu.VMEM((ROW_W,), jnp.float32),
            pltpu.SemaphoreType.DMA,
            pltpu.SemaphoreType.DMA,
            pltpu.SemaphoreType.DMA,
        ],
    )
    def sc_kernel(mem_hbm, scr_sc_hbm, scr_tc_hbm, w_hbm, out_hbm,
                  idx_v, rows_v, w_v, srow_v, obuf0, obuf1,
                  gsem, osem0, osem1):
        wid = lax.axis_index("s") * NC + lax.axis_index("c")
        base = wid * bpw

        pltpu.sync_copy(w_hbm, w_v.at[pl.ds(0, N)])

        def stage_idx(r, carry):
            pltpu.async_copy(mem_hbm.at[base + r],
                             idx_v.at[r, pl.ds(0, N)], gsem)
            return carry

        lax.fori_loop(0, bpw, stage_idx, 0)

        def stage_drain(r, carry):
            pltpu.make_async_copy(mem_hbm.at[0],
                                  idx_v.at[0, pl.ds(0, N)], gsem).wait()
            return carry

        lax.fori_loop(0, bpw, stage_drain, 0)

        iota = lax.iota(jnp.int32, L)

        def gather_row(r, carry):
            v0 = idx_v[r, pl.ds(0, L)]
            v1 = idx_v[r, pl.ds(L, L)]
            for k in range(N):
                row = v0[k] if k < L else v1[k - L]
                slot = r * N + k

                @pl.when(row < WSC)
                def _from_sc():
                    idx0 = iota * WSC + row
                    pltpu.async_copy(scr_sc_hbm.at[idx0],
                                     rows_v.at[slot, pl.ds(0, L)], gsem)
                    pltpu.async_copy(scr_sc_hbm.at[idx0 + (L * WSC)],
                                     rows_v.at[slot, pl.ds(L, L)], gsem)

                @pl.when(row >= WSC)
                def _from_tc():
                    lcl = row - WSC
                    off = (((lcl >> 13) << 18) + ((lcl & 2047) << 7)
                           + (((lcl >> 11) & 3) << 5))
                    off = pl.multiple_of(off, 32)
                    pltpu.async_copy(scr_tc_hbm.at[pl.ds(off, D)],
                                     rows_v.at[slot], gsem)
            return carry

        lax.fori_loop(0, bpw, gather_row, 0)

        def gather_drain(r, carry):
            for k in range(N):
                pltpu.make_async_copy(scr_tc_hbm.at[pl.ds(0, D)],
                                      rows_v.at[0], gsem).wait()
            return carry

        lax.fori_loop(0, bpw, gather_drain, 0)

        wv0 = w_v[pl.ds(0, L)]
        wv1 = w_v[pl.ds(L, L)]
        w_sc = [wv0[i] if i < L else wv1[i - L] for i in range(N)]

        obufs = (obuf0, obuf1)
        osems = (osem0, osem1)

        def row_pair_body(t, carry):
            for b in range(2):
                r = t * 2 + b
                obuf = obufs[b]
                osem = osems[b]
                lbase = r * N

                @pl.when(t > 0)
                def _wait_prev():
                    pltpu.make_async_copy(
                        obuf, out_hbm.at[0], osem).wait()

                for i in range(N):
                    for h in range(2):
                        sl = pl.ds(h * L, L)
                        v = rows_v[lbase + i, sl]
                        obuf[pl.ds(i * D + h * L, L)] = v
                        srow_v[i, sl] = v * w_sc[i]

                p = 0
                for i in range(N):
                    si0 = srow_v[i, pl.ds(0, L)]
                    si1 = srow_v[i, pl.ds(L, L)]
                    for j in range(i + 1, N):
                        o = (N + p) * D
                        obuf[pl.ds(o, L)] = si0 * srow_v[j, pl.ds(0, L)]
                        obuf[pl.ds(o + L, L)] = si1 * srow_v[j, pl.ds(L, L)]
                        p += 1

                pltpu.async_copy(obuf, out_hbm.at[base + r], osem)
            return carry

        lax.fori_loop(0, bpw // 2, row_pair_body, 0)

        pltpu.make_async_copy(obuf0, out_hbm.at[0], osem0).wait()
        pltpu.make_async_copy(obuf1, out_hbm.at[0], osem1).wait()

    return sc_kernel


def kernel(user, memory, item_table, weights):
    del user
    batch = memory.shape[0]
    v = item_table.shape[0]
    table_t = item_table.T
    scr_sc = _build_sc_detile(v)(table_t)
    scr_tc = _build_tc_detile(v)(table_t).reshape(-1)
    sc_kernel = _build_main(batch)
    return sc_kernel(memory.astype(jnp.int32), scr_sc, scr_tc, weights)
